# Initial kernel scaffold; baseline (speedup 1.0000x reference)
import jax
import jax.numpy as jnp
from jax.experimental import pallas as pl


def kernel(hidden_states, input_ids, attention_mask):
    raise NotImplementedError("write your pallas kernel here")


# speedup vs baseline: 1.7261x; 1.7261x over previous
import functools

import jax
import jax.numpy as jnp
from jax.experimental import pallas as pl
from jax.experimental.pallas import tpu as pltpu

SEP = 2
CH = 64
NBUF = 8

M_N = 0
M_NT = 1
M_TEND = 2
M_LSTART = 3
M_LEND = 4
M_TVALID = 5
M_LVALID = 6


def _bounds_kernel(ids_ref, mask_ref, meta_ref, ktot_ref):
    ids = ids_ref[...]
    msk = mask_ref[...]
    B, S = ids.shape
    eq = (ids == SEP)
    idx = jax.lax.broadcasted_iota(jnp.int32, (B, S), 1)

    def first_pos(cond):
        big = jnp.where(cond, idx, S)
        m = jnp.min(big, axis=1, keepdims=True)
        return jnp.where(m == S, 0, m).astype(jnp.int32)

    pos1 = first_pos(eq)
    pos2 = first_pos(eq & (idx > pos1))
    pos3 = first_pos(eq & (idx > pos2))
    total = jnp.sum(eq.astype(jnp.int32), axis=1, keepdims=True)
    mask_sum = jnp.sum(msk, axis=1, keepdims=True).astype(jnp.int32)
    has2 = total >= 2
    has3 = total >= 3

    title_end = pos1
    lead_start = pos2 + 1
    lead_end = jnp.where(has3, pos3, mask_sum)

    title_cnt = jnp.maximum(title_end - 1, 0)
    lead_cnt = jnp.maximum(lead_end - lead_start, 0)
    t_valid = (has2 & (title_cnt > 0)).astype(jnp.int32)
    l_valid = (has2 & (lead_cnt > 0)).astype(jnp.int32)

    te_eff = jnp.where(has2, title_end, 0)
    n_t = jnp.maximum((te_eff + (CH - 1)) // CH, 1)
    lstart_a = (lead_start // 8) * 8
    n_l = jnp.where(has2 & (lead_cnt > 0),
                    (lead_end - lstart_a + (CH - 1)) // CH, 0)
    n = n_t + n_l

    meta = jnp.concatenate(
        [n, n_t, title_end, lead_start, lead_end, t_valid, l_valid,
         jnp.zeros_like(n)], axis=1)
    meta_ref[...] = meta
    ktot_ref[...] = jnp.sum(n).reshape(1, 1)


def _pool_kernel(meta_ref, ktot_ref, h_ref, title_ref, lead_ref,
                 buf_ref, acc_ref, fb_ref, sems):
    S = h_ref.shape[1]
    H = h_ref.shape[2]
    ktot = ktot_ref[0, 0]

    def chunk_window(b, i):
        n_t = meta_ref[b, M_NT]
        tend = meta_ref[b, M_TEND]
        lstart = meta_ref[b, M_LSTART]
        lend = meta_ref[b, M_LEND]
        is_title = i < n_t
        lstart_a = (lstart // 8) * 8
        base = jnp.where(is_title, i * CH, lstart_a + (i - n_t) * CH)
        lo = jnp.where(is_title, jnp.maximum(base, 1),
                       jnp.maximum(base, lstart))
        hi = jnp.minimum(jnp.where(is_title, tend, lend), base + CH)
        r0 = pl.multiple_of(jnp.minimum(base, S - CH), 8)
        return r0, lo, hi, is_title

    def start_copy(slot, b, i):
        r0, _, _, _ = chunk_window(b, i)
        pltpu.make_async_copy(
            h_ref.at[b, pl.ds(r0, CH), :],
            buf_ref.at[slot],
            sems.at[slot],
        ).start()

    B = h_ref.shape[0]

    def advance(b, i):
        nb = meta_ref[b, M_N]
        last = i + 1 >= nb
        return (jnp.where(last, jnp.minimum(b + 1, B - 1), b),
                jnp.where(last, 0, i + 1))

    def pro_body(j, st):
        ib, ii = st

        @pl.when(j < ktot)
        def _():
            start_copy(j % NBUF, ib, ii)

        return advance(ib, ii)

    ib, ii = jax.lax.fori_loop(0, NBUF, pro_body,
                               (jnp.int32(0), jnp.int32(0)))

    def body(k, st):
        ib, ii, cb, ci = st
        slot = k % NBUF
        pltpu.make_async_copy(
            h_ref.at[cb, pl.ds(0, CH), :], buf_ref.at[slot], sems.at[slot]
        ).wait()

        r0, lo, hi, is_title = chunk_window(cb, ci)
        buf = buf_ref[slot]
        ridx = r0 + jax.lax.broadcasted_iota(jnp.int32, (CH, 1), 0)
        w = ((ridx >= lo) & (ridx < hi)).astype(jnp.float32)
        s = jnp.sum(buf * w, axis=0, keepdims=True)

        @pl.when(ci == 0)
        def _():
            fb_ref[...] = buf[0:1, :]
            acc_ref[...] = jnp.zeros_like(acc_ref)

        @pl.when(is_title)
        def _():
            acc_ref[0:1, :] += s

        @pl.when(jnp.logical_not(is_title))
        def _():
            acc_ref[1:2, :] += s

        @pl.when(ci + 1 >= meta_ref[cb, M_N])
        def _():
            tend = meta_ref[cb, M_TEND]
            lstart = meta_ref[cb, M_LSTART]
            lend = meta_ref[cb, M_LEND]
            t_cnt = jnp.maximum(tend - 1, 0).astype(jnp.float32)
            l_cnt = jnp.maximum(lend - lstart, 0).astype(jnp.float32)
            fb = fb_ref[...]
            t_mean = acc_ref[0:1, :] / jnp.maximum(t_cnt, 1.0)
            l_mean = acc_ref[1:2, :] / jnp.maximum(l_cnt, 1.0)
            t_out = jnp.where(meta_ref[cb, M_TVALID] > 0, t_mean, fb)
            l_out = jnp.where(meta_ref[cb, M_LVALID] > 0, l_mean, fb)
            title_ref[pl.ds(cb, 1), :] = t_out
            lead_ref[pl.ds(cb, 1), :] = l_out

        @pl.when(k + NBUF < ktot)
        def _():
            start_copy(slot, ib, ii)

        ib2, ii2 = advance(ib, ii)
        cb2, ci2 = advance(cb, ci)
        return ib2, ii2, cb2, ci2

    jax.lax.fori_loop(0, ktot, body,
                      (ib, ii, jnp.int32(0), jnp.int32(0)))


@jax.jit
def kernel(hidden_states, input_ids, attention_mask):
    B, S, H = hidden_states.shape
    ids = input_ids.astype(jnp.int32)
    msk = attention_mask.astype(jnp.int32)

    meta, ktot = pl.pallas_call(
        _bounds_kernel,
        out_shape=[
            jax.ShapeDtypeStruct((B, 8), jnp.int32),
            jax.ShapeDtypeStruct((1, 1), jnp.int32),
        ],
    )(ids, msk)

    title, lead = pl.pallas_call(
        _pool_kernel,
        in_specs=[
            pl.BlockSpec(memory_space=pltpu.SMEM),
            pl.BlockSpec(memory_space=pltpu.SMEM),
            pl.BlockSpec(memory_space=pl.ANY),
        ],
        out_specs=[
            pl.BlockSpec(memory_space=pltpu.VMEM),
            pl.BlockSpec(memory_space=pltpu.VMEM),
        ],
        out_shape=[
            jax.ShapeDtypeStruct((B, H), jnp.float32),
            jax.ShapeDtypeStruct((B, H), jnp.float32),
        ],
        scratch_shapes=[
            pltpu.VMEM((NBUF, CH, H), jnp.float32),
            pltpu.VMEM((2, H), jnp.float32),
            pltpu.VMEM((1, H), jnp.float32),
            pltpu.SemaphoreType.DMA((NBUF,)),
        ],
    )(meta, ktot, hidden_states)
    return title, lead
